# baseline (speedup 1.0000x reference)
import jax
import jax.numpy as jnp
from jax.experimental import pallas as pl
from jax.experimental.pallas import tpu as pltpu

_TM = 2048
_KS = 2


def _router_body(x_ref, w_ref, b_ref, o_ref, wbf_ref):
    k = pl.program_id(1)
    kh = x_ref.shape[1]

    @pl.when((pl.program_id(0) == 0) & (k == 0))
    def _cache_w():
        wbf_ref[...] = w_ref[...].astype(jnp.bfloat16)

    part = jax.lax.dot_general(
        x_ref[...].astype(jnp.bfloat16),
        wbf_ref[:, pl.ds(k * kh, kh)],
        dimension_numbers=(((1,), (1,)), ((), ())),
        preferred_element_type=jnp.float32,
    )

    @pl.when(k == 0)
    def _first():
        o_ref[...] = part + b_ref[...]

    @pl.when(k == 1)
    def _second():
        o_ref[...] += part


def kernel(hidden_states, W, b):
    tokens, hidden = hidden_states.shape
    experts = W.shape[0]
    kh = hidden // _KS
    b2 = b.reshape(1, experts)
    return pl.pallas_call(
        _router_body,
        grid=(tokens // _TM, _KS),
        in_specs=[
            pl.BlockSpec((_TM, kh), lambda i, k: (i, k)),
            pl.BlockSpec((experts, hidden), lambda i, k: (0, 0)),
            pl.BlockSpec((1, experts), lambda i, k: (0, 0)),
        ],
        out_specs=pl.BlockSpec((_TM, experts), lambda i, k: (i, 0)),
        out_shape=jax.ShapeDtypeStruct((tokens, experts), jnp.float32),
        scratch_shapes=[pltpu.VMEM((experts, hidden), jnp.bfloat16)],
    )(hidden_states, W, b2)


# speedup vs baseline: 1.0769x; 1.0769x over previous
import jax
import jax.numpy as jnp
from jax.experimental import pallas as pl
from jax.experimental.pallas import tpu as pltpu

_TM = 1024


def _router_body(xl_ref, xr_ref, w_ref, b_ref, o_ref):
    kh = xl_ref.shape[1]
    wb = w_ref[...].astype(jnp.bfloat16)
    dn = (((1,), (1,)), ((), ()))
    accl = jax.lax.dot_general(
        xl_ref[...].astype(jnp.bfloat16), wb[:, :kh],
        dimension_numbers=dn, preferred_element_type=jnp.float32)
    accr = jax.lax.dot_general(
        xr_ref[...].astype(jnp.bfloat16), wb[:, kh:],
        dimension_numbers=dn, preferred_element_type=jnp.float32)
    o_ref[...] = accl + accr + b_ref[...]


def kernel(hidden_states, W, b):
    tokens, hidden = hidden_states.shape
    experts = W.shape[0]
    kh = hidden // 2
    b2 = b.reshape(1, experts)
    return pl.pallas_call(
        _router_body,
        grid=(tokens // _TM,),
        in_specs=[
            pl.BlockSpec((_TM, kh), lambda i: (i, 0)),
            pl.BlockSpec((_TM, kh), lambda i: (i, 1)),
            pl.BlockSpec((experts, hidden), lambda i: (0, 0)),
            pl.BlockSpec((1, experts), lambda i: (0, 0)),
        ],
        out_specs=pl.BlockSpec((_TM, experts), lambda i: (i, 0)),
        out_shape=jax.ShapeDtypeStruct((tokens, experts), jnp.float32),
        compiler_params=pltpu.CompilerParams(
            dimension_semantics=("parallel",),
        ),
    )(hidden_states, hidden_states, W, b2)
